# baseline (speedup 1.0000x reference)
import functools
import jax
import jax.numpy as jnp
from jax import lax
from jax.experimental import pallas as pl
from jax.experimental.pallas import tpu as pltpu
from jax.experimental.pallas import tpu_sc as plsc

TOKENS = 16384
DIM = 256
VOCAB = 8192
TT = 1024
CHUNK = 2048
NT = TOKENS // TT
NCH = VOCAB // CHUNK
NG = CHUNK // 128
W1, W2 = 2736, 5472
G1, L1 = W1 // 128, W1 % 128
G2, L2 = W2 // 128, W2 % 128


SB = 128


def _vq_argmin_kernel(zb_ref, z2_ref, cb_ref, c2_ref, idx_ref, md_ref):
    lane = lax.broadcasted_iota(jnp.int32, (SB, 128), 1)
    inf = jnp.full((SB, 128), jnp.inf, jnp.float32)
    zero_g = jnp.zeros((SB, 128), jnp.int32)

    def upd(rv, rg, dg, G):
        take = dg < rv
        return jnp.where(take, dg, rv), jnp.where(take, G, rg)

    def finalize(rv, rg):
        m = jnp.min(rv, axis=1, keepdims=True)
        j = rg * 128 + lane
        jm = jnp.min(jnp.where(rv == m, j, VOCAB), axis=1, keepdims=True)
        return m, jm

    md_total = jnp.zeros((), jnp.float32)
    for s in range(TT // SB):
        zbs = zb_ref[pl.ds(s * SB, SB), :]
        z2s = z2_ref[pl.ds(s * SB, SB), :]
        wval, widx = [], []
        rv, rg = inf, zero_g
        for k in range(NCH):
            cbc = cb_ref[pl.ds(k * CHUNK, CHUNK), :]
            t = lax.dot_general(
                zbs, cbc, (((1,), (1,)), ((), ())),
                preferred_element_type=jnp.float32,
            )
            for g in range(NG):
                G = k * NG + g
                tg = t[:, g * 128:(g + 1) * 128]
                dg = (z2s + c2_ref[:, pl.ds(G * 128, 128)]) - tg
                if G in (G1, G2):
                    lb = L1 if G == G1 else L2
                    rv, rg = upd(rv, rg, jnp.where(lane < lb, dg, jnp.inf), G)
                    v, i = finalize(rv, rg)
                    wval.append(v); widx.append(i)
                    rv, rg = upd(inf, zero_g,
                                 jnp.where(lane >= lb, dg, jnp.inf), G)
                else:
                    rv, rg = upd(rv, rg, dg, G)
        v, i = finalize(rv, rg)
        wval.append(v); widx.append(i)

        V = wval[0].astype(jnp.bfloat16).astype(jnp.float32)
        I = widx[0]
        P = wval[0]
        for w in (1, 2):
            a_lt = V < wval[w]
            a_eq = V == wval[w]
            keep = a_lt | (a_eq & (I < widx[w]))
            I = jnp.where(keep, I, widx[w])
            P = jnp.where(keep, P, wval[w])
            V = jnp.where(a_lt, V, wval[w]).astype(
                jnp.bfloat16).astype(jnp.float32)

        idx_ref[0, 0, pl.ds(s * SB, SB)] = I.reshape(SB)
        md_total = md_total + jnp.sum(P)
    md_ref[...] = md_total.reshape(1, 1, 1)


@functools.lru_cache(maxsize=4)
def _vq_argmin_call(n_tokens):
    nt = n_tokens // TT

    @jax.jit
    def f(zb, z2, cb_bf, c2):
        return _vq_argmin_pallas(nt)(zb, z2, cb_bf, c2)

    return f


def _vq_argmin_pallas(nt):
    return pl.pallas_call(
        _vq_argmin_kernel,
        grid=(nt,),
        in_specs=[
            pl.BlockSpec((TT, DIM), lambda i: (i, 0)),
            pl.BlockSpec((TT, 1), lambda i: (i, 0)),
            pl.BlockSpec((VOCAB, DIM), lambda i: (0, 0)),
            pl.BlockSpec((1, VOCAB), lambda i: (0, 0)),
        ],
        out_specs=[
            pl.BlockSpec((1, 1, TT), lambda i: (i, 0, 0)),
            pl.BlockSpec((1, 1, 1), lambda i: (i, 0, 0)),
        ],
        out_shape=[
            jax.ShapeDtypeStruct((nt, 1, TT), jnp.int32),
            jax.ShapeDtypeStruct((nt, 1, 1), jnp.float32),
        ],
        compiler_params=pltpu.CompilerParams(
            dimension_semantics=("arbitrary",),
        ),
    )


@functools.lru_cache(maxsize=4)
def _sc_gather(n_tokens):
    info = plsc.get_sparse_core_info()
    nw = info.num_cores * info.num_subcores
    b_per_w = n_tokens // nw
    cs = 128
    nchunks = b_per_w // cs
    mesh = plsc.VectorSubcoreMesh(core_axis_name="c", subcore_axis_name="s")

    @functools.partial(
        pl.kernel, mesh=mesh,
        out_type=jax.ShapeDtypeStruct((n_tokens, DIM), jnp.float32),
        scratch_types=[
            [pltpu.VMEM((cs,), jnp.int32) for _ in range(nchunks)],
            [pltpu.VMEM((cs, DIM), jnp.float32) for _ in range(2)],
            [pltpu.SemaphoreType.DMA for _ in range(2)],
        ],
    )
    def k(table_hbm, idx_hbm, out_hbm, idx_v, bufs, sems):
        wid = lax.axis_index("s") * info.num_cores + lax.axis_index("c")
        base = wid * b_per_w
        for c in range(nchunks):
            pltpu.sync_copy(idx_hbm.at[pl.ds(base + c * cs, cs)], idx_v[c])
        cps = [None, None]
        cps[0] = pltpu.async_copy(table_hbm.at[idx_v[0]], bufs[0], sems[0])
        for c in range(nchunks):
            cps[c % 2].wait()
            if c + 1 < nchunks:
                cps[(c + 1) % 2] = pltpu.async_copy(
                    table_hbm.at[idx_v[c + 1]], bufs[(c + 1) % 2], sems[(c + 1) % 2])
            pltpu.sync_copy(bufs[c % 2], out_hbm.at[pl.ds(base + c * cs, cs)])

    return k


def kernel(x, codebook):
    b, c, h, w = x.shape
    z_e = jnp.transpose(x, (0, 2, 3, 1))
    z_flat = z_e.reshape(-1, c)
    z2 = jnp.sum(z_flat ** 2, axis=1, keepdims=True)
    c2 = jnp.sum(codebook ** 2, axis=1).reshape(1, VOCAB)
    zb = z_flat.astype(jnp.bfloat16) * jnp.bfloat16(2.0)
    cb_bf = codebook.astype(jnp.bfloat16)
    idx_t, md = _vq_argmin_call(TOKENS)(zb, z2, cb_bf, c2)
    codebook_indices = idx_t.reshape(TOKENS)
    z_q = _sc_gather(TOKENS)(codebook, codebook_indices)
    loss = 2.0 * jnp.sum(md) / (TOKENS * DIM)
    z_q_out = jnp.transpose(z_q.reshape(b, h, w, c), (0, 3, 1, 2))
    return (z_q_out, codebook_indices, loss)


# speedup vs baseline: 1.2567x; 1.2567x over previous
import functools
import jax
import jax.numpy as jnp
from jax import lax
from jax.experimental import pallas as pl
from jax.experimental.pallas import tpu as pltpu
from jax.experimental.pallas import tpu_sc as plsc

TOKENS = 16384
DIM = 256
VOCAB = 8192
TT = 1024
CHUNK = 2048
NT = TOKENS // TT
NCH = VOCAB // CHUNK
NG = CHUNK // 128
W1, W2 = 2736, 5472
G1, L1 = W1 // 128, W1 % 128
G2, L2 = W2 // 128, W2 % 128


def _vq_argmin_kernel(zb_ref, z2_ref, cb_ref, c2_ref, idx_ref, md_ref):
    zb = zb_ref[...]
    z2 = z2_ref[...]
    lane = lax.broadcasted_iota(jnp.int32, (TT, 128), 1)
    inf = jnp.full((TT, 128), jnp.inf, jnp.float32)
    zero_g = jnp.zeros((TT, 128), jnp.int32)

    def upd(rv, rg, dg, G):
        take = dg < rv
        return jnp.where(take, dg, rv), jnp.where(take, G, rg)

    def finalize(rv, rg):
        m = jnp.min(rv, axis=1, keepdims=True)
        j = rg * 128 + lane
        jm = jnp.min(jnp.where(rv == m, j, VOCAB), axis=1, keepdims=True)
        return m, jm

    wval, widx = [], []
    rv, rg = inf, zero_g
    for k in range(NCH):
        cbc = cb_ref[pl.ds(k * CHUNK, CHUNK), :]
        t = lax.dot_general(
            zb, cbc, (((1,), (1,)), ((), ())),
            preferred_element_type=jnp.float32,
        )
        for g in range(NG):
            G = k * NG + g
            tg = t[:, g * 128:(g + 1) * 128]
            dg = (z2 + c2_ref[:, pl.ds(G * 128, 128)]) - tg
            if G in (G1, G2):
                lb = L1 if G == G1 else L2
                rv, rg = upd(rv, rg, jnp.where(lane < lb, dg, jnp.inf), G)
                v, i = finalize(rv, rg)
                wval.append(v); widx.append(i)
                rv, rg = upd(inf, zero_g, jnp.where(lane >= lb, dg, jnp.inf), G)
            else:
                rv, rg = upd(rv, rg, dg, G)
    v, i = finalize(rv, rg)
    wval.append(v); widx.append(i)

    V = wval[0].astype(jnp.bfloat16).astype(jnp.float32)
    I = widx[0]
    P = wval[0]
    for w in (1, 2):
        a_lt = V < wval[w]
        a_eq = V == wval[w]
        keep = a_lt | (a_eq & (I < widx[w]))
        I = jnp.where(keep, I, widx[w])
        P = jnp.where(keep, P, wval[w])
        V = jnp.where(a_lt, V, wval[w]).astype(jnp.bfloat16).astype(jnp.float32)

    idx_ref[...] = I.reshape(1, 1, TT)
    md_ref[...] = jnp.sum(P).reshape(1, 1, 1)


@functools.lru_cache(maxsize=4)
def _vq_argmin_call(n_tokens):
    nt = n_tokens // TT

    @jax.jit
    def f(zb, z2, cb_bf, c2):
        return _vq_argmin_pallas(nt)(zb, z2, cb_bf, c2)

    return f


def _vq_argmin_pallas(nt):
    return pl.pallas_call(
        _vq_argmin_kernel,
        grid=(nt,),
        in_specs=[
            pl.BlockSpec((TT, DIM), lambda i: (i, 0)),
            pl.BlockSpec((TT, 1), lambda i: (i, 0)),
            pl.BlockSpec((VOCAB, DIM), lambda i: (0, 0)),
            pl.BlockSpec((1, VOCAB), lambda i: (0, 0)),
        ],
        out_specs=[
            pl.BlockSpec((1, 1, TT), lambda i: (i, 0, 0)),
            pl.BlockSpec((1, 1, 1), lambda i: (i, 0, 0)),
        ],
        out_shape=[
            jax.ShapeDtypeStruct((nt, 1, TT), jnp.int32),
            jax.ShapeDtypeStruct((nt, 1, 1), jnp.float32),
        ],
        compiler_params=pltpu.CompilerParams(
            dimension_semantics=("arbitrary",),
        ),
    )


@functools.lru_cache(maxsize=4)
def _sc_gather(n_tokens):
    info = plsc.get_sparse_core_info()
    nw = info.num_cores * info.num_subcores
    b_per_w = n_tokens // nw
    cs = 128
    nchunks = b_per_w // cs
    mesh = plsc.VectorSubcoreMesh(core_axis_name="c", subcore_axis_name="s")

    @functools.partial(
        pl.kernel, mesh=mesh,
        out_type=jax.ShapeDtypeStruct((n_tokens, DIM), jnp.float32),
        scratch_types=[
            [pltpu.VMEM((cs,), jnp.int32) for _ in range(nchunks)],
            [pltpu.VMEM((cs, DIM), jnp.float32) for _ in range(2)],
            [pltpu.SemaphoreType.DMA for _ in range(2)],
        ],
    )
    def k(table_hbm, idx_hbm, out_hbm, idx_v, bufs, sems):
        wid = lax.axis_index("s") * info.num_cores + lax.axis_index("c")
        base = wid * b_per_w
        for c in range(nchunks):
            pltpu.sync_copy(idx_hbm.at[pl.ds(base + c * cs, cs)], idx_v[c])
        cps = [None, None]
        cps[0] = pltpu.async_copy(table_hbm.at[idx_v[0]], bufs[0], sems[0])
        for c in range(nchunks):
            cps[c % 2].wait()
            if c + 1 < nchunks:
                cps[(c + 1) % 2] = pltpu.async_copy(
                    table_hbm.at[idx_v[c + 1]], bufs[(c + 1) % 2], sems[(c + 1) % 2])
            pltpu.sync_copy(bufs[c % 2], out_hbm.at[pl.ds(base + c * cs, cs)])

    return k


def kernel(x, codebook):
    b, c, h, w = x.shape
    z_e = jnp.transpose(x, (0, 2, 3, 1))
    z_flat = z_e.reshape(-1, c)
    z2 = jnp.sum(z_flat ** 2, axis=1, keepdims=True)
    c2 = jnp.sum(codebook ** 2, axis=1).reshape(1, VOCAB)
    zb = z_flat.astype(jnp.bfloat16) * jnp.bfloat16(2.0)
    cb_bf = codebook.astype(jnp.bfloat16)
    idx_t, md = _vq_argmin_call(TOKENS)(zb, z2, cb_bf, c2)
    codebook_indices = idx_t.reshape(TOKENS)
    z_q = _sc_gather(TOKENS)(codebook, codebook_indices)
    loss = 2.0 * jnp.sum(md) / (TOKENS * DIM)
    z_q_out = jnp.transpose(z_q.reshape(b, h, w, c), (0, 3, 1, 2))
    return (z_q_out, codebook_indices, loss)
